# baseline (speedup 1.0000x reference)
import jax
import jax.numpy as jnp
from jax.experimental import pallas as pl

_PART = (0, 0, 0, 0,
         1, 1, 1, 2,
         2, 2, 2, 2, 2, 2,
         3, 3, 3, 3, 3, 3,
         4, 4, 4, 4, 4, 4)

def _expand_kernel(in_ref, out_ref):
    b = in_ref[...]
    out_ref[...] = jnp.concatenate([b[..., p:p + 1] for p in _PART], axis=-1)


def kernel(body):
    N, C, T, P = body.shape
    J = 26
    flat = body.reshape(N * C, T, P)
    B = 8
    grid = (flat.shape[0] // B,)
    out = pl.pallas_call(
        _expand_kernel,
        grid=grid,
        in_specs=[pl.BlockSpec((B, T, P), lambda i: (i, 0, 0))],
        out_specs=pl.BlockSpec((B, T, J), lambda i: (i, 0, 0)),
        out_shape=jax.ShapeDtypeStruct((N * C, T, J), body.dtype),
    )(flat)
    return out.reshape(N, C, T, J)
